# baseline (speedup 1.0000x reference)
import functools

import jax
import jax.numpy as jnp
from jax.experimental import pallas as pl
from jax.experimental.pallas import tpu as pltpu

ALPHA = 0.2
NEG = -9e15


def _prep1_body(x_ref, w_ref, an_ref, h_ref, hb_ref, ncol_ref):
    h = jnp.dot(x_ref[...], w_ref[...], preferred_element_type=jnp.float32)
    h_ref[...] = h
    hb_ref[...] = h.astype(jnp.bfloat16)
    ncol_ref[...] = jnp.dot(h, an_ref[...], preferred_element_type=jnp.float32)


def _softmax_num_denom(adj, m, s, nrow):
    dense = (s + nrow) * m
    dense = jnp.maximum(dense, ALPHA * dense)
    p = jnp.exp(dense) * adj
    denom = jnp.sum(p, axis=1, keepdims=True)
    return p, denom


NPIECE = 5


def _attn1_body(*refs):
    (a0, a1, a2, a3, a4, m0, m1, m2, m3, m4, hb_ref, hrows_ref,
     as_ref, nrow_ref, w2_ref, an2_ref,
     h2_ref, n2row_ref, mm_ref, h2e_ref) = refs
    adj_refs = (a0, a1, a2, a3, a4)
    m_refs = (m0, m1, m2, m3, m4)
    s = jnp.dot(hrows_ref[...], as_ref[...], preferred_element_type=jnp.float32)
    nrow = nrow_ref[...]
    hb = hb_ref[...]
    pr = a0.shape[0]
    hp = None
    denoms = []
    for k in range(NPIECE):
        adj = adj_refs[k][...]
        m = m_refs[k][...]
        mm_ref[k * pr:(k + 1) * pr, :] = (
            jnp.where(adj > 0, m, -1.0).astype(jnp.bfloat16))
        p, d = _softmax_num_denom(adj, m, s[k * pr:(k + 1) * pr], nrow)
        part = jnp.dot(p.astype(jnp.bfloat16), hb,
                       preferred_element_type=jnp.float32)
        hp = part if hp is None else jnp.concatenate([hp, part], axis=0)
        denoms.append(d)
    denom = jnp.concatenate(denoms, axis=0)
    hp = hp / denom
    hp = jnp.where(hp > 0, hp, jnp.exp(hp) - 1.0)
    h2 = jnp.dot(hp, w2_ref[...], preferred_element_type=jnp.float32)
    h2_ref[...] = h2
    n2row_ref[...] = jnp.dot(h2, an2_ref[...], preferred_element_type=jnp.float32)
    rb = h2.shape[0]
    h2e_ref[:, :16] = h2.astype(jnp.bfloat16)
    h2e_ref[:, 16:] = jnp.ones((rb, 8), jnp.bfloat16)


def _attn2_body(mm_ref, hfull_ref, hrows_ref, as_ref, nrow_ref,
                c_ref, z_ref, q_ref):
    mm = mm_ref[...]
    s = jnp.dot(hrows_ref[...], as_ref[...], preferred_element_type=jnp.float32)
    b = s.astype(jnp.bfloat16) + nrow_ref[...]
    dense = b * mm
    dense = jnp.maximum(dense, jnp.bfloat16(ALPHA) * dense)
    p = jnp.where(mm >= 0, jnp.exp(dense), jnp.bfloat16(0.0))
    hpe = jnp.dot(p, hfull_ref[...], preferred_element_type=jnp.float32)
    hp = hpe[:, :16]
    denom = hpe[:, 16:17]
    hp = hp / denom
    hp = jnp.where(hp > 0, hp, jnp.exp(hp) - 1.0)
    norm = jnp.sqrt(jnp.sum(hp * hp, axis=1, keepdims=True))
    z = hp / jnp.maximum(norm, 1e-12)
    z_ref[...] = z
    c = c_ref[...]
    zn = jnp.sum(z * z, axis=1, keepdims=True)
    cn = jnp.sum(c * c, axis=1, keepdims=True).T
    cross = jax.lax.dot_general(z, c, (((1,), (1,)), ((), ())),
                                preferred_element_type=jnp.float32)
    dist2 = zn + cn - 2.0 * cross
    qv = 1.0 / (1.0 + dist2)
    q_ref[...] = qv / jnp.sum(qv, axis=1, keepdims=True)


def _apred_body(zrows_ref, zfull_ref, out_ref):
    g = jax.lax.dot_general(zrows_ref[...], zfull_ref[...],
                            (((1,), (1,)), ((), ())),
                            preferred_element_type=jnp.float32)
    out_ref[...] = jax.nn.sigmoid(g)


@functools.partial(jax.jit, static_argnums=())
def kernel(x, adj, M, W1, a_self1, a_neighs1, W2, a_self2, a_neighs2, cluster):
    N, D = x.shape
    H = W1.shape[1]
    E = W2.shape[1]
    K = cluster.shape[0]
    f32 = jnp.float32

    RBP = 1000
    h1, h1b, n1row = pl.pallas_call(
        _prep1_body,
        grid=(N // RBP,),
        in_specs=[
            pl.BlockSpec((RBP, D), lambda i: (i, 0)),
            pl.BlockSpec((D, H), lambda i: (0, 0)),
            pl.BlockSpec((H, 1), lambda i: (0, 0)),
        ],
        out_specs=[
            pl.BlockSpec((RBP, H), lambda i: (i, 0)),
            pl.BlockSpec((RBP, H), lambda i: (i, 0)),
            pl.BlockSpec((RBP, 1), lambda i: (i, 0)),
        ],
        out_shape=[
            jax.ShapeDtypeStruct((N, H), f32),
            jax.ShapeDtypeStruct((N, H), jnp.bfloat16),
            jax.ShapeDtypeStruct((N, 1), f32),
        ],
        compiler_params=pltpu.CompilerParams(
            dimension_semantics=("parallel",)),
    )(x, W1, a_neighs1)
    n1row = n1row.T

    RB = 200
    PR = RB // NPIECE
    stripe_specs = [
        pl.BlockSpec((PR, N), (lambda i, k=k: (NPIECE * i + k, 0)))
        for k in range(NPIECE)
    ]
    h2, n2row, Mm, h2e = pl.pallas_call(
        _attn1_body,
        grid=(N // RB,),
        in_specs=stripe_specs + stripe_specs + [
            pl.BlockSpec((N, H), lambda i: (0, 0)),
            pl.BlockSpec((RB, H), lambda i: (i, 0)),
            pl.BlockSpec((H, 1), lambda i: (0, 0)),
            pl.BlockSpec((1, N), lambda i: (0, 0)),
            pl.BlockSpec((H, E), lambda i: (0, 0)),
            pl.BlockSpec((E, 1), lambda i: (0, 0)),
        ],
        out_specs=[
            pl.BlockSpec((RB, E), lambda i: (i, 0)),
            pl.BlockSpec((RB, 1), lambda i: (i, 0)),
            pl.BlockSpec((RB, N), lambda i: (i, 0)),
            pl.BlockSpec((RB, E + 8), lambda i: (i, 0)),
        ],
        out_shape=[
            jax.ShapeDtypeStruct((N, E), f32),
            jax.ShapeDtypeStruct((N, 1), f32),
            jax.ShapeDtypeStruct((N, N), jnp.bfloat16),
            jax.ShapeDtypeStruct((N, E + 8), jnp.bfloat16),
        ],
        compiler_params=pltpu.CompilerParams(
            dimension_semantics=("parallel",)),
    )(adj, adj, adj, adj, adj, M, M, M, M, M,
      h1b, h1, a_self1, n1row, W2, a_neighs2)
    n2row = n2row.T.astype(jnp.bfloat16)

    RB2 = 1000
    z, q = pl.pallas_call(
        _attn2_body,
        grid=(N // RB2,),
        in_specs=[
            pl.BlockSpec((RB2, N), lambda i: (i, 0)),
            pl.BlockSpec((N, E + 8), lambda i: (0, 0)),
            pl.BlockSpec((RB2, E), lambda i: (i, 0)),
            pl.BlockSpec((E, 1), lambda i: (0, 0)),
            pl.BlockSpec((1, N), lambda i: (0, 0)),
            pl.BlockSpec((K, E), lambda i: (0, 0)),
        ],
        out_specs=[
            pl.BlockSpec((RB2, E), lambda i: (i, 0)),
            pl.BlockSpec((RB2, K), lambda i: (i, 0)),
        ],
        out_shape=[
            jax.ShapeDtypeStruct((N, E), f32),
            jax.ShapeDtypeStruct((N, K), f32),
        ],
        compiler_params=pltpu.CompilerParams(
            dimension_semantics=("parallel",)),
    )(Mm, h2e, h2, a_self2, n2row, cluster)

    RBA = 1000
    a_pred = pl.pallas_call(
        _apred_body,
        grid=(N // RBA,),
        in_specs=[
            pl.BlockSpec((RBA, E), lambda i: (i, 0)),
            pl.BlockSpec((N, E), lambda i: (0, 0)),
        ],
        out_specs=pl.BlockSpec((RBA, N), lambda i: (i, 0)),
        out_shape=jax.ShapeDtypeStruct((N, N), f32),
        compiler_params=pltpu.CompilerParams(
            dimension_semantics=("parallel",)),
    )(z, z)

    return (a_pred, z, q)


# speedup vs baseline: 1.0484x; 1.0484x over previous
import functools

import jax
import jax.numpy as jnp
from jax.experimental import pallas as pl
from jax.experimental.pallas import tpu as pltpu

ALPHA = 0.2
NEG = -9e15


def _prep1_body(x_ref, w_ref, an_ref, h_ref, hb_ref, ncol_ref):
    h = jnp.dot(x_ref[...], w_ref[...], preferred_element_type=jnp.float32)
    h_ref[...] = h
    hb_ref[...] = h.astype(jnp.bfloat16)
    ncol_ref[...] = jnp.dot(h, an_ref[...], preferred_element_type=jnp.float32)


def _softmax_num_denom(adj, m, s, nrow):
    dense = (s + nrow) * m
    dense = jnp.maximum(dense, ALPHA * dense)
    p = jnp.exp(dense) * adj
    denom = jnp.sum(p, axis=1, keepdims=True)
    return p, denom


def _attn1_body(adj_ref, m_ref, hb_ref, hrows_ref, as_ref, nrow_ref,
                w2_ref, an2_ref, h2_ref, n2row_ref, mm_ref, h2e_ref):
    adj = adj_ref[...]
    m = m_ref[...]
    mm_ref[...] = jnp.where(
        adj > 0, jnp.round(m * 252.0) - 126.0, -128.0).astype(jnp.int8)
    s = jnp.dot(hrows_ref[...], as_ref[...], preferred_element_type=jnp.float32)
    p, denom = _softmax_num_denom(adj, m, s, nrow_ref[...])
    hp = jnp.dot(p.astype(jnp.bfloat16), hb_ref[...],
                 preferred_element_type=jnp.float32)
    hp = hp / denom
    hp = jnp.where(hp > 0, hp, jnp.exp(hp) - 1.0)
    h2 = jnp.dot(hp, w2_ref[...], preferred_element_type=jnp.float32)
    h2_ref[...] = h2
    n2row_ref[...] = jnp.dot(h2, an2_ref[...], preferred_element_type=jnp.float32)
    rb = h2.shape[0]
    h2e_ref[:, :16] = h2.astype(jnp.bfloat16)
    h2e_ref[:, 16:] = jnp.ones((rb, 8), jnp.bfloat16)


def _attn2_body(mm_ref, hfull_ref, hrows_ref, as_ref, nrow_ref,
                c_ref, z_ref, q_ref):
    v = mm_ref[...].astype(jnp.bfloat16)
    s = jnp.dot(hrows_ref[...], as_ref[...], preferred_element_type=jnp.float32)
    b = s.astype(jnp.bfloat16) + nrow_ref[...]
    dense = b * (v + jnp.bfloat16(126.0))
    dense = jnp.maximum(dense, jnp.bfloat16(ALPHA) * dense)
    p = jnp.where(v > jnp.bfloat16(-127.0), jnp.exp(dense), jnp.bfloat16(0.0))
    hpe = jnp.dot(p, hfull_ref[...], preferred_element_type=jnp.float32)
    hp = hpe[:, :16]
    denom = hpe[:, 16:17]
    hp = hp / denom
    hp = jnp.where(hp > 0, hp, jnp.exp(hp) - 1.0)
    norm = jnp.sqrt(jnp.sum(hp * hp, axis=1, keepdims=True))
    z = hp / jnp.maximum(norm, 1e-12)
    z_ref[...] = z
    c = c_ref[...]
    zn = jnp.sum(z * z, axis=1, keepdims=True)
    cn = jnp.sum(c * c, axis=1, keepdims=True).T
    cross = jax.lax.dot_general(z, c, (((1,), (1,)), ((), ())),
                                preferred_element_type=jnp.float32)
    dist2 = zn + cn - 2.0 * cross
    qv = 1.0 / (1.0 + dist2)
    q_ref[...] = qv / jnp.sum(qv, axis=1, keepdims=True)


def _apred_body(zrows_ref, zfull_ref, out_ref):
    g = jax.lax.dot_general(zrows_ref[...], zfull_ref[...],
                            (((1,), (1,)), ((), ())),
                            preferred_element_type=jnp.float32)
    out_ref[...] = jax.nn.sigmoid(g)


@functools.partial(jax.jit, static_argnums=())
def kernel(x, adj, M, W1, a_self1, a_neighs1, W2, a_self2, a_neighs2, cluster):
    N, D = x.shape
    H = W1.shape[1]
    E = W2.shape[1]
    K = cluster.shape[0]
    f32 = jnp.float32

    RBP = 1000
    h1, h1b, n1row = pl.pallas_call(
        _prep1_body,
        grid=(N // RBP,),
        in_specs=[
            pl.BlockSpec((RBP, D), lambda i: (i, 0)),
            pl.BlockSpec((D, H), lambda i: (0, 0)),
            pl.BlockSpec((H, 1), lambda i: (0, 0)),
        ],
        out_specs=[
            pl.BlockSpec((RBP, H), lambda i: (i, 0)),
            pl.BlockSpec((RBP, H), lambda i: (i, 0)),
            pl.BlockSpec((RBP, 1), lambda i: (i, 0)),
        ],
        out_shape=[
            jax.ShapeDtypeStruct((N, H), f32),
            jax.ShapeDtypeStruct((N, H), jnp.bfloat16),
            jax.ShapeDtypeStruct((N, 1), f32),
        ],
        compiler_params=pltpu.CompilerParams(
            dimension_semantics=("parallel",)),
    )(x, W1, a_neighs1)
    n1row = n1row.T

    RB = 200
    h2, n2row, Mm, h2e = pl.pallas_call(
        _attn1_body,
        grid=(N // RB,),
        in_specs=[
            pl.BlockSpec((RB, N), lambda i: (i, 0)),
            pl.BlockSpec((RB, N), lambda i: (i, 0)),
            pl.BlockSpec((N, H), lambda i: (0, 0)),
            pl.BlockSpec((RB, H), lambda i: (i, 0)),
            pl.BlockSpec((H, 1), lambda i: (0, 0)),
            pl.BlockSpec((1, N), lambda i: (0, 0)),
            pl.BlockSpec((H, E), lambda i: (0, 0)),
            pl.BlockSpec((E, 1), lambda i: (0, 0)),
        ],
        out_specs=[
            pl.BlockSpec((RB, E), lambda i: (i, 0)),
            pl.BlockSpec((RB, 1), lambda i: (i, 0)),
            pl.BlockSpec((RB, N), lambda i: (i, 0)),
            pl.BlockSpec((RB, E + 8), lambda i: (i, 0)),
        ],
        out_shape=[
            jax.ShapeDtypeStruct((N, E), f32),
            jax.ShapeDtypeStruct((N, 1), f32),
            jax.ShapeDtypeStruct((N, N), jnp.int8),
            jax.ShapeDtypeStruct((N, E + 8), jnp.bfloat16),
        ],
        compiler_params=pltpu.CompilerParams(
            dimension_semantics=("parallel",)),
    )(adj, M, h1b, h1, a_self1, n1row, W2, a_neighs2)
    n2row = (n2row.T * (1.0 / 252.0)).astype(jnp.bfloat16)
    a_self2s = a_self2 * (1.0 / 252.0)

    RB2 = 1000
    z, q = pl.pallas_call(
        _attn2_body,
        grid=(N // RB2,),
        in_specs=[
            pl.BlockSpec((RB2, N), lambda i: (i, 0)),
            pl.BlockSpec((N, E + 8), lambda i: (0, 0)),
            pl.BlockSpec((RB2, E), lambda i: (i, 0)),
            pl.BlockSpec((E, 1), lambda i: (0, 0)),
            pl.BlockSpec((1, N), lambda i: (0, 0)),
            pl.BlockSpec((K, E), lambda i: (0, 0)),
        ],
        out_specs=[
            pl.BlockSpec((RB2, E), lambda i: (i, 0)),
            pl.BlockSpec((RB2, K), lambda i: (i, 0)),
        ],
        out_shape=[
            jax.ShapeDtypeStruct((N, E), f32),
            jax.ShapeDtypeStruct((N, K), f32),
        ],
        compiler_params=pltpu.CompilerParams(
            dimension_semantics=("parallel",)),
    )(Mm, h2e, h2, a_self2s, n2row, cluster)

    RBA = 1000
    a_pred = pl.pallas_call(
        _apred_body,
        grid=(N // RBA,),
        in_specs=[
            pl.BlockSpec((RBA, E), lambda i: (i, 0)),
            pl.BlockSpec((N, E), lambda i: (0, 0)),
        ],
        out_specs=pl.BlockSpec((RBA, N), lambda i: (i, 0)),
        out_shape=jax.ShapeDtypeStruct((N, N), f32),
        compiler_params=pltpu.CompilerParams(
            dimension_semantics=("parallel",)),
    )(z, z)

    return (a_pred, z, q)


# speedup vs baseline: 1.1394x; 1.0868x over previous
import functools

import jax
import jax.numpy as jnp
from jax.experimental import pallas as pl
from jax.experimental.pallas import tpu as pltpu

ALPHA = 0.2


def _prep1_body(x_ref, w_ref, an_ref, hb_ref, ncol_ref):
    h = jnp.dot(x_ref[...], w_ref[...], preferred_element_type=jnp.float32)
    hb_ref[...] = h.astype(jnp.bfloat16)
    ncol_ref[...] = jnp.dot(h, an_ref[...], preferred_element_type=jnp.float32)


def _attn1_body(adj_ref, m_ref, hb_ref, as_ref, nrow_ref,
                w2_ref, an2_ref, comb_ref, mm_ref):
    adj = adj_ref[...]
    m = m_ref[...]
    mm_ref[...] = jnp.where(adj > 0, m, -1.0).astype(jnp.bfloat16)
    rb = adj.shape[0]
    hrows = hb_ref[pl.ds(pl.program_id(0) * rb, rb), :]
    s = jnp.dot(hrows, as_ref[...], preferred_element_type=jnp.float32)
    dense = (s + nrow_ref[...]) * m
    dense = jnp.maximum(dense, ALPHA * dense)
    p = jnp.exp(dense) * adj
    denom = jnp.sum(p, axis=1, keepdims=True)
    hp = jnp.dot(p.astype(jnp.bfloat16), hb_ref[...],
                 preferred_element_type=jnp.float32)
    hp = hp / denom
    hp = jnp.where(hp > 0, hp, jnp.exp(hp) - 1.0)
    h2 = jnp.dot(hp, w2_ref[...], preferred_element_type=jnp.float32)
    comb_ref[:, :16] = h2
    comb_ref[:, 16:17] = jnp.dot(h2, an2_ref[...],
                                 preferred_element_type=jnp.float32)
    comb_ref[:, 17:] = jnp.zeros((rb, 7), jnp.float32)


def _attn2_body(mm_ref, comb_ref, as_ref, nrow_ref, c_ref, z_ref, q_ref):
    mm = mm_ref[...]
    rb = mm.shape[0]
    comb = comb_ref[...]
    hrows = comb_ref[pl.ds(pl.program_id(0) * rb, rb), :16]
    s = jnp.dot(hrows, as_ref[...], preferred_element_type=jnp.float32)
    b = s.astype(jnp.bfloat16) + nrow_ref[...]
    dense = b * mm
    dense = jnp.maximum(dense, jnp.bfloat16(ALPHA) * dense)
    p = jnp.where(mm >= 0, jnp.exp(dense), jnp.bfloat16(0.0))
    nfull = comb.shape[0]
    h2e = jnp.concatenate(
        [comb[:, :16].astype(jnp.bfloat16),
         jnp.ones((nfull, 8), jnp.bfloat16)], axis=1)
    hpe = jnp.dot(p, h2e, preferred_element_type=jnp.float32)
    hp = hpe[:, :16]
    denom = hpe[:, 16:17]
    hp = hp / denom
    hp = jnp.where(hp > 0, hp, jnp.exp(hp) - 1.0)
    norm = jnp.sqrt(jnp.sum(hp * hp, axis=1, keepdims=True))
    z = hp / jnp.maximum(norm, 1e-12)
    z_ref[...] = z
    c = c_ref[...]
    zn = jnp.sum(z * z, axis=1, keepdims=True)
    cn = jnp.sum(c * c, axis=1, keepdims=True).T
    cross = jax.lax.dot_general(z, c, (((1,), (1,)), ((), ())),
                                preferred_element_type=jnp.float32)
    dist2 = zn + cn - 2.0 * cross
    qv = 1.0 / (1.0 + dist2)
    q_ref[...] = qv / jnp.sum(qv, axis=1, keepdims=True)


def _apred_body(zfull_ref, out_ref):
    rb = out_ref.shape[0]
    zrows = zfull_ref[pl.ds(pl.program_id(0) * rb, rb), :]
    g = jax.lax.dot_general(zrows, zfull_ref[...],
                            (((1,), (1,)), ((), ())),
                            preferred_element_type=jnp.float32)
    out_ref[...] = jax.nn.sigmoid(g)


@functools.partial(jax.jit, static_argnums=())
def kernel(x, adj, M, W1, a_self1, a_neighs1, W2, a_self2, a_neighs2, cluster):
    N, D = x.shape
    H = W1.shape[1]
    E = W2.shape[1]
    K = cluster.shape[0]
    f32 = jnp.float32
    bf16 = jnp.bfloat16

    RBP = 1000
    h1b, n1row = pl.pallas_call(
        _prep1_body,
        grid=(N // RBP,),
        in_specs=[
            pl.BlockSpec((RBP, D), lambda i: (i, 0)),
            pl.BlockSpec((D, H), lambda i: (0, 0)),
            pl.BlockSpec((H, 1), lambda i: (0, 0)),
        ],
        out_specs=[
            pl.BlockSpec((RBP, H), lambda i: (i, 0)),
            pl.BlockSpec((RBP, 1), lambda i: (i, 0)),
        ],
        out_shape=[
            jax.ShapeDtypeStruct((N, H), bf16),
            jax.ShapeDtypeStruct((N, 1), f32),
        ],
        compiler_params=pltpu.CompilerParams(
            dimension_semantics=("parallel",)),
    )(x, W1, a_neighs1)
    n1row = n1row.T

    RB = 200
    comb, Mm = pl.pallas_call(
        _attn1_body,
        grid=(N // RB,),
        in_specs=[
            pl.BlockSpec((RB, N), lambda i: (i, 0)),
            pl.BlockSpec((RB, N), lambda i: (i, 0)),
            pl.BlockSpec((N, H), lambda i: (0, 0)),
            pl.BlockSpec((H, 1), lambda i: (0, 0)),
            pl.BlockSpec((1, N), lambda i: (0, 0)),
            pl.BlockSpec((H, E), lambda i: (0, 0)),
            pl.BlockSpec((E, 1), lambda i: (0, 0)),
        ],
        out_specs=[
            pl.BlockSpec((RB, E + 8), lambda i: (i, 0)),
            pl.BlockSpec((RB, N), lambda i: (i, 0)),
        ],
        out_shape=[
            jax.ShapeDtypeStruct((N, E + 8), f32),
            jax.ShapeDtypeStruct((N, N), bf16),
        ],
        compiler_params=pltpu.CompilerParams(
            dimension_semantics=("parallel",)),
    )(adj, M, h1b, a_self1.astype(bf16), n1row, W2, a_neighs2)
    n2row = comb[:, 16:17].T.astype(bf16)

    RB2 = 1000
    z, q = pl.pallas_call(
        _attn2_body,
        grid=(N // RB2,),
        in_specs=[
            pl.BlockSpec((RB2, N), lambda i: (i, 0)),
            pl.BlockSpec((N, E + 8), lambda i: (0, 0)),
            pl.BlockSpec((E, 1), lambda i: (0, 0)),
            pl.BlockSpec((1, N), lambda i: (0, 0)),
            pl.BlockSpec((K, E), lambda i: (0, 0)),
        ],
        out_specs=[
            pl.BlockSpec((RB2, E), lambda i: (i, 0)),
            pl.BlockSpec((RB2, K), lambda i: (i, 0)),
        ],
        out_shape=[
            jax.ShapeDtypeStruct((N, E), f32),
            jax.ShapeDtypeStruct((N, K), f32),
        ],
        compiler_params=pltpu.CompilerParams(
            dimension_semantics=("parallel",)),
    )(Mm, comb, a_self2, n2row, cluster)

    RBA = 1000
    a_pred = pl.pallas_call(
        _apred_body,
        grid=(N // RBA,),
        in_specs=[
            pl.BlockSpec((N, E), lambda i: (0, 0)),
        ],
        out_specs=pl.BlockSpec((RBA, N), lambda i: (i, 0)),
        out_shape=jax.ShapeDtypeStruct((N, N), f32),
        compiler_params=pltpu.CompilerParams(
            dimension_semantics=("parallel",)),
    )(z)

    return (a_pred, z, q)
